# Initial kernel scaffold; baseline (speedup 1.0000x reference)
import jax
import jax.numpy as jnp
from jax.experimental import pallas as pl


def kernel(x, edge_index):
    raise NotImplementedError("write your pallas kernel here")


# speedup vs baseline: 5.7779x; 5.7779x over previous
import jax
import jax.numpy as jnp
from jax import lax
from jax.experimental import pallas as pl
from jax.experimental.pallas import tpu as pltpu
from jax.experimental.pallas import tpu_sc as plsc

_N = 10000
_NP = 10240
_E = 160000
_EP = 163840
_D = 256
_H = 64
_Q = 4
_NC = 2
_NS = 16
_NPASS = _Q // _NC
_CH = 128
_TCH = _EP // _NS // _CH
_DCH = _EP // (_NS * _NC) // _CH
_STR = _NP // _NS
_RB = 512
_ROWS = _Q * _NP
_GRID = _ROWS // _RB

_mesh = plsc.VectorSubcoreMesh(core_axis_name="c", subcore_axis_name="s")
_sc_params = pltpu.CompilerParams(use_tc_tiling_on_sc=False)


def _deg_body(rows_hbm, deg_out, idx_v, ones_v, zbuf_v, acc_sh):
    c = lax.axis_index("c")
    s = lax.axis_index("s")
    w = s * _NC + c

    def _fill_ones(i, _):
        ones_v[pl.ds(i * 16, 16)] = jnp.ones((16,), jnp.float32)
        return 0

    lax.fori_loop(0, _CH // 16, _fill_ones, 0)

    def _fill_zero(i, _):
        zbuf_v[pl.ds(i * 16, 16)] = jnp.zeros((16,), jnp.float32)
        return 0

    lax.fori_loop(0, _STR // 16, _fill_zero, 0)
    pltpu.sync_copy(zbuf_v, acc_sh.at[pl.ds(s * _STR, _STR)])
    pltpu.sync_copy(rows_hbm.at[w], idx_v)
    plsc.subcore_barrier()

    def _scatter(j, _):
        pltpu.sync_copy(ones_v, acc_sh.at[idx_v.at[j]], add=True)
        return 0

    lax.fori_loop(0, _DCH, _scatter, 0)
    plsc.subcore_barrier()
    pltpu.sync_copy(
        acc_sh.at[pl.ds(s * _STR, _STR)],
        deg_out.at[pl.ds(c * _NP + s * _STR, _STR)],
    )


def _deg_call(rows_deg):
    return pl.kernel(
        _deg_body,
        out_type=jax.ShapeDtypeStruct((_NC * _NP,), jnp.float32),
        mesh=_mesh,
        compiler_params=_sc_params,
        scratch_types=[
            pltpu.VMEM((_DCH, _CH), jnp.int32),
            pltpu.VMEM((_CH,), jnp.float32),
            pltpu.VMEM((_STR,), jnp.float32),
            pltpu.VMEM_SHARED((_NP,), jnp.float32),
        ],
    )(rows_deg)


def _prop_body(u_hbm, rows_hbm, cols_hbm, t_out,
               rows_v, cols_v, offs_v, g0, g1, sem0, sem1, acc_sh):
    c = lax.axis_index("c")
    s = lax.axis_index("s")
    pltpu.sync_copy(rows_hbm.at[s], rows_v)
    pltpu.sync_copy(cols_hbm.at[s], cols_v)

    def _fill_zero(i, _):
        for l in range(_H // 16):
            g0[i, pl.ds(l * 16, 16)] = jnp.zeros((16,), jnp.float32)
        return 0

    lax.fori_loop(0, _CH, _fill_zero, 0)

    for p in range(_NPASS):
        base = (c * _NPASS + p) * _NP

        def _offset(j, _):
            for l in range(_CH // 16):
                offs_v[j, pl.ds(l * 16, 16)] = (
                    cols_v[j, pl.ds(l * 16, 16)] + base
                )
            return 0

        lax.fori_loop(0, _TCH, _offset, 0)
        for k in range(_STR // _CH):
            pltpu.sync_copy(g0, acc_sh.at[pl.ds(s * _STR + k * _CH, _CH)])
        plsc.subcore_barrier()

        def _edge_chunks(j, _):
            d0 = pltpu.async_copy(u_hbm.at[offs_v.at[2 * j]], g0, sem0)
            d1 = pltpu.async_copy(u_hbm.at[offs_v.at[2 * j + 1]], g1, sem1)
            d0.wait()
            pltpu.sync_copy(g0, acc_sh.at[rows_v.at[2 * j]], add=True)
            d1.wait()
            pltpu.sync_copy(g1, acc_sh.at[rows_v.at[2 * j + 1]], add=True)
            return 0

        lax.fori_loop(0, _TCH // 2, _edge_chunks, 0)
        plsc.subcore_barrier()
        pltpu.sync_copy(
            acc_sh.at[pl.ds(s * _STR, _STR)],
            t_out.at[pl.ds(base + s * _STR, _STR)],
        )
        if p + 1 < _NPASS:
            lax.fori_loop(0, _CH, _fill_zero, 0)

    return None


def _prop_call(u, rows_m, cols_m):
    return pl.kernel(
        _prop_body,
        out_type=jax.ShapeDtypeStruct((_ROWS, _H), jnp.float32),
        mesh=_mesh,
        compiler_params=_sc_params,
        scratch_types=[
            pltpu.VMEM((_TCH, _CH), jnp.int32),
            pltpu.VMEM((_TCH, _CH), jnp.int32),
            pltpu.VMEM((_TCH, _CH), jnp.int32),
            pltpu.VMEM((_CH, _H), jnp.float32),
            pltpu.VMEM((_CH, _H), jnp.float32),
            pltpu.SemaphoreType.DMA,
            pltpu.SemaphoreType.DMA,
            pltpu.VMEM_SHARED((_NP, _H), jnp.float32),
        ],
    )(u, rows_m, cols_m)


def _rsqrt_body(parts_ref, s_ref, s2_ref):
    deg = parts_ref[0:1, :] + parts_ref[1:2, :]
    s = jnp.where(deg > 0.0, lax.rsqrt(deg), jnp.zeros_like(deg))
    s2 = s * s
    for q in range(_Q):
        s_ref[q:q + 1, :] = s
        s2_ref[q:q + 1, :] = s2


def _rsqrt_call(deg_parts):
    return pl.pallas_call(
        _rsqrt_body,
        out_shape=[jax.ShapeDtypeStruct((_Q, _NP), jnp.float32)] * 2,
    )(deg_parts)


def _row_spec():
    return pl.BlockSpec((_RB, _H), lambda i: (i, 0))


def _col_spec():
    return pl.BlockSpec((_RB, 1), lambda i: (i, 0))


def _u0_body(x_ref, s_ref, u_ref):
    u_ref[...] = s_ref[...] * x_ref[...]


def _u0_call(x_flat, s_col):
    return pl.pallas_call(
        _u0_body,
        grid=(_GRID,),
        in_specs=[_row_spec(), _col_spec()],
        out_specs=_row_spec(),
        out_shape=jax.ShapeDtypeStruct((_ROWS, _H), jnp.float32),
    )(x_flat, s_col)


def _scale1_body(t_ref, s_ref, s2_ref, u_ref, a_ref):
    t = t_ref[...]
    u_ref[...] = s2_ref[...] * t
    a_ref[...] = s_ref[...] * t


def _scale1_call(t, s_col, s2_col):
    return pl.pallas_call(
        _scale1_body,
        grid=(_GRID,),
        in_specs=[_row_spec(), _col_spec(), _col_spec()],
        out_specs=[_row_spec()] * 2,
        out_shape=[jax.ShapeDtypeStruct((_ROWS, _H), jnp.float32)] * 2,
    )(t, s_col, s2_col)


def _scale2_body(t_ref, s_ref, s2_ref, acc_ref, u_ref, a_ref):
    t = t_ref[...]
    u_ref[...] = s2_ref[...] * t
    a_ref[...] = acc_ref[...] + s_ref[...] * t


def _scale2_call(t, s_col, s2_col, acc):
    return pl.pallas_call(
        _scale2_body,
        grid=(_GRID,),
        in_specs=[_row_spec(), _col_spec(), _col_spec(), _row_spec()],
        out_specs=[_row_spec()] * 2,
        out_shape=[jax.ShapeDtypeStruct((_ROWS, _H), jnp.float32)] * 2,
    )(t, s_col, s2_col, acc)


def _final_body(t_ref, s_ref, acc_ref, x_ref, o_ref):
    o_ref[...] = 0.25 * (x_ref[...] + acc_ref[...] + s_ref[...] * t_ref[...])


def _final_call(t, s_col, acc, x_flat):
    return pl.pallas_call(
        _final_body,
        grid=(_GRID,),
        in_specs=[_row_spec(), _col_spec(), _row_spec(), _row_spec()],
        out_specs=_row_spec(),
        out_shape=jax.ShapeDtypeStruct((_ROWS, _H), jnp.float32),
    )(t, s_col, acc, x_flat)


def kernel(x, edge_index):
    rows = edge_index[0]
    cols = edge_index[1]
    pad = _N + (jnp.arange(_EP - _E, dtype=jnp.int32) % (_NP - _N))
    rows_p = jnp.concatenate([rows, pad])
    cols_p = jnp.concatenate([cols, pad])
    rows_deg = rows_p.reshape(_NS * _NC, _DCH, _CH)
    rows_m = rows_p.reshape(_NS, _TCH, _CH)
    cols_m = cols_p.reshape(_NS, _TCH, _CH)
    x_flat = (
        jnp.pad(x, ((0, _NP - _N), (0, 0)))
        .reshape(_NP, _Q, _H)
        .transpose(1, 0, 2)
        .reshape(_ROWS, _H)
    )

    deg_parts = _deg_call(rows_deg).reshape(_NC, _NP)
    s2d, s22d = _rsqrt_call(deg_parts)
    s_col = s2d.reshape(_ROWS, 1)
    s2_col = s22d.reshape(_ROWS, 1)

    u = _u0_call(x_flat, s_col)
    t = _prop_call(u, rows_m, cols_m)
    u, acc = _scale1_call(t, s_col, s2_col)
    t = _prop_call(u, rows_m, cols_m)
    u, acc = _scale2_call(t, s_col, s2_col, acc)
    t = _prop_call(u, rows_m, cols_m)
    out_flat = _final_call(t, s_col, acc, x_flat)

    return (
        out_flat.reshape(_Q, _NP, _H)
        .transpose(1, 0, 2)
        .reshape(_NP, _D)[:_N]
    )


# speedup vs baseline: 10.4323x; 1.8055x over previous
import jax
import jax.numpy as jnp
from jax import lax
from jax.experimental import pallas as pl
from jax.experimental.pallas import tpu as pltpu
from jax.experimental.pallas import tpu_sc as plsc

_N = 10000
_NP = 10240
_E = 160000
_EP = 163840
_D = 256
_H = 64
_Q = 4
_NC = 2
_NS = 16
_NPASS = _Q // _NC
_K = 3
_CH = 128
_TCH = _EP // _NS // _CH
_STR = _NP // _NS
_ESEG = _STR // 4
_ROWS = _Q * _NP
_NBUF = 4

_mesh = plsc.VectorSubcoreMesh(core_axis_name="c", subcore_axis_name="s")
_sc_params = pltpu.CompilerParams(use_tc_tiling_on_sc=False)


def _newton_rsqrt(d):
    y = lax.bitcast_convert_type(
        jnp.int32(0x5F3759DF) - lax.shift_right_logical(
            lax.bitcast_convert_type(d, jnp.int32), jnp.int32(1)),
        jnp.float32)
    for _ in range(3):
        y = y * (1.5 - 0.5 * d * y * y)
    return jnp.where(d > 0.0, y, jnp.zeros_like(y))


def _fused_body(x_hbm, rows_hbm, cols_hbm,
                out_hbm, u_hbm,
                rows_v, cols_v,
                g0, g1, g2, g3, gs0, gs1, gs2, gs3,
                ss0, ss1, ss2, ss3,
                ones_v, sv, s2v, ebuf, tbuf,
                acc_sh, deg_sh):
    c = lax.axis_index("c")
    s = lax.axis_index("s")
    g = (g0, g1, g2, g3)
    gsem = (gs0, gs1, gs2, gs3)
    ssem = (ss0, ss1, ss2, ss3)
    n_it = _TCH // _NBUF

    pltpu.sync_copy(rows_hbm.at[s], rows_v)
    pltpu.sync_copy(cols_hbm.at[s], cols_v)

    def _fill_ones(i, _):
        ones_v[pl.ds(i * 16, 16)] = jnp.ones((16,), jnp.float32)
        return 0

    lax.fori_loop(0, _CH // 16, _fill_ones, 0)

    def _fill_zero1(i, _):
        sv[pl.ds(i * 16, 16)] = jnp.zeros((16,), jnp.float32)
        return 0

    lax.fori_loop(0, _STR // 16, _fill_zero1, 0)

    def _fill_zero(i, _):
        for l in range(_H // 16):
            g0[i, pl.ds(l * 16, 16)] = jnp.zeros((16,), jnp.float32)
        return 0

    lax.fori_loop(0, _CH, _fill_zero, 0)

    pltpu.sync_copy(sv, deg_sh.at[pl.ds(s * _STR, _STR)])
    plsc.subcore_barrier()

    def _deg_scatter(j, _):
        pltpu.sync_copy(ones_v, deg_sh.at[rows_v.at[j]], add=True)
        return 0

    lax.fori_loop(0, _TCH, _deg_scatter, 0)
    plsc.subcore_barrier()

    pltpu.sync_copy(deg_sh.at[pl.ds(s * _STR, _STR)], sv)

    def _rsqrt_stripe(i, _):
        d = sv[pl.ds(i * 16, 16)]
        y = _newton_rsqrt(d)
        sv[pl.ds(i * 16, 16)] = y
        s2v[pl.ds(i * 16, 16)] = y * y
        return 0

    lax.fori_loop(0, _STR // 16, _rsqrt_stripe, 0)

    for p in range(_NPASS):
        qbase = (c * _NPASS + p) * _NP + s * _STR
        for qs in range(4):
            seg = qs * _ESEG
            pltpu.sync_copy(x_hbm.at[pl.ds(qbase + seg, _ESEG)], ebuf)

            def _scale_u0(i, _):
                s16 = sv[pl.ds(seg + i * 16, 16)]
                for r in range(16):
                    f = s16[r]
                    for l in range(_H // 16):
                        sl = pl.ds(l * 16, 16)
                        ebuf[i * 16 + r, sl] = f * ebuf[i * 16 + r, sl]
                return 0

            lax.fori_loop(0, _ESEG // 16, _scale_u0, 0)
            pltpu.sync_copy(ebuf, u_hbm.at[pl.ds(qbase + seg, _ESEG)])

    plsc.subcore_barrier()

    for k in range(_K):
        for p in range(_NPASS):
            base = (c * _NPASS + p) * _NP
            if k == 0 and p == 0:
                delta = c * (_NPASS * _NP)
            elif p == 0:
                delta = -_NP
            else:
                delta = _NP

            def _offset(j, _):
                for l in range(_CH // 16):
                    sl = pl.ds(l * 16, 16)
                    cols_v[j, sl] = cols_v[j, sl] + delta
                return 0

            lax.fori_loop(0, _TCH, _offset, 0)
            for z in range(_STR // _CH):
                pltpu.sync_copy(
                    g0, acc_sh.at[pl.ds(s * _STR + z * _CH, _CH)])
            plsc.subcore_barrier()

            for b in range(_NBUF):
                pltpu.async_copy(u_hbm.at[cols_v.at[b]], g[b], gsem[b])

            def _edge_chunks(j, _):
                for b in range(_NBUF):
                    pltpu.make_async_copy(
                        u_hbm.at[cols_v.at[_NBUF * j + b]], g[b], gsem[b]
                    ).wait()
                    pltpu.async_copy(
                        g[b], acc_sh.at[rows_v.at[_NBUF * j + b]], ssem[b],
                        add=True)

                @pl.when(j < n_it - 1)
                def _prefetch():
                    for b in range(_NBUF):
                        pltpu.make_async_copy(
                            g[b], acc_sh.at[rows_v.at[_NBUF * j + b]],
                            ssem[b]).wait()
                        pltpu.async_copy(
                            u_hbm.at[cols_v.at[_NBUF * (j + 1) + b]], g[b],
                            gsem[b])

                return 0

            lax.fori_loop(0, n_it, _edge_chunks, 0)
            for b in range(_NBUF):
                pltpu.make_async_copy(
                    g[b], acc_sh.at[rows_v.at[_TCH - _NBUF + b]], ssem[b]
                ).wait()
            plsc.subcore_barrier()

            hbase = base + s * _STR
            for qs in range(4):
                seg = qs * _ESEG
                pltpu.sync_copy(
                    acc_sh.at[pl.ds(s * _STR + seg, _ESEG)], tbuf)
                if k == 0:
                    pltpu.sync_copy(
                        x_hbm.at[pl.ds(hbase + seg, _ESEG)], ebuf)
                else:
                    pltpu.sync_copy(
                        out_hbm.at[pl.ds(hbase + seg, _ESEG)], ebuf)

                if k + 1 < _K:
                    def _scale_mid(i, _):
                        s16 = sv[pl.ds(seg + i * 16, 16)]
                        s216 = s2v[pl.ds(seg + i * 16, 16)]
                        for r in range(16):
                            f = s16[r]
                            f2 = s216[r]
                            for l in range(_H // 16):
                                sl = pl.ds(l * 16, 16)
                                t_il = tbuf[i * 16 + r, sl]
                                ebuf[i * 16 + r, sl] = (
                                    ebuf[i * 16 + r, sl] + f * t_il)
                                tbuf[i * 16 + r, sl] = f2 * t_il
                        return 0

                    lax.fori_loop(0, _ESEG // 16, _scale_mid, 0)
                    pltpu.sync_copy(
                        ebuf, out_hbm.at[pl.ds(hbase + seg, _ESEG)])
                    pltpu.sync_copy(
                        tbuf, u_hbm.at[pl.ds(hbase + seg, _ESEG)])
                else:
                    def _scale_last(i, _):
                        s16 = sv[pl.ds(seg + i * 16, 16)]
                        for r in range(16):
                            f = s16[r]
                            for l in range(_H // 16):
                                sl = pl.ds(l * 16, 16)
                                ebuf[i * 16 + r, sl] = 0.25 * (
                                    ebuf[i * 16 + r, sl]
                                    + f * tbuf[i * 16 + r, sl])
                        return 0

                    lax.fori_loop(0, _ESEG // 16, _scale_last, 0)
                    pltpu.sync_copy(
                        ebuf, out_hbm.at[pl.ds(hbase + seg, _ESEG)])

            if k + 1 < _K or p + 1 < _NPASS:
                lax.fori_loop(0, _CH, _fill_zero, 0)
                plsc.subcore_barrier()


def _fused_call(x_flat, rows_m, cols_m):
    return pl.kernel(
        _fused_body,
        out_type=[
            jax.ShapeDtypeStruct((_ROWS, _H), jnp.float32),
            jax.ShapeDtypeStruct((_ROWS, _H), jnp.float32),
        ],
        mesh=_mesh,
        compiler_params=_sc_params,
        scratch_types=[
            pltpu.VMEM((_TCH, _CH), jnp.int32),
            pltpu.VMEM((_TCH, _CH), jnp.int32),
        ] + [pltpu.VMEM((_CH, _H), jnp.float32)] * _NBUF
        + [pltpu.SemaphoreType.DMA] * (2 * _NBUF)
        + [
            pltpu.VMEM((_CH,), jnp.float32),
            pltpu.VMEM((_STR,), jnp.float32),
            pltpu.VMEM((_STR,), jnp.float32),
            pltpu.VMEM((_ESEG, _H), jnp.float32),
            pltpu.VMEM((_ESEG, _H), jnp.float32),
            pltpu.VMEM_SHARED((_NP, _H), jnp.float32),
            pltpu.VMEM_SHARED((_NP,), jnp.float32),
        ],
    )(x_flat, rows_m, cols_m)


def kernel(x, edge_index):
    rows = edge_index[0]
    cols = edge_index[1]
    pad = _N + (jnp.arange(_EP - _E, dtype=jnp.int32) % (_NP - _N))
    rows_p = jnp.concatenate([rows, pad])
    cols_p = jnp.concatenate([cols, pad])
    rows_m = rows_p.reshape(_NS, _TCH, _CH)
    cols_m = cols_p.reshape(_NS, _TCH, _CH)
    x_flat = (
        jnp.pad(x, ((0, _NP - _N), (0, 0)))
        .reshape(_NP, _Q, _H)
        .transpose(1, 0, 2)
        .reshape(_ROWS, _H)
    )
    out_flat, _ = _fused_call(x_flat, rows_m, cols_m)
    return (
        out_flat.reshape(_Q, _NP, _H)
        .transpose(1, 0, 2)
        .reshape(_NP, _D)[:_N]
    )


# speedup vs baseline: 10.9255x; 1.0473x over previous
import jax
import jax.numpy as jnp
from jax import lax
from jax.experimental import pallas as pl
from jax.experimental.pallas import tpu as pltpu
from jax.experimental.pallas import tpu_sc as plsc

_N = 10000
_NP = 10240
_E = 160000
_EP = 163840
_D = 256
_H = 64
_Q = 4
_NC = 2
_NS = 16
_NPASS = _Q // _NC
_K = 3
_CH = 128
_TCH = _EP // _NS // _CH
_STR = _NP // _NS
_ESEG = 80
_NSEG = _STR // _ESEG
_ROWS = _Q * _NP
_NBUF = 4

_mesh = plsc.VectorSubcoreMesh(core_axis_name="c", subcore_axis_name="s")
_sc_params = pltpu.CompilerParams(use_tc_tiling_on_sc=False)


def _newton_rsqrt(d):
    y = lax.bitcast_convert_type(
        jnp.int32(0x5F3759DF) - lax.shift_right_logical(
            lax.bitcast_convert_type(d, jnp.int32), jnp.int32(1)),
        jnp.float32)
    for _ in range(3):
        y = y * (1.5 - 0.5 * d * y * y)
    return jnp.where(d > 0.0, y, jnp.zeros_like(y))


def _fused_body(x_hbm, rows_hbm, cols_hbm,
                out_hbm, u_hbm,
                rows_v, cols_v,
                g0, g1, g2, g3,
                gs0, gs1, gs2, gs3,
                ss0, ss1, ss2, ss3,
                ones_v, sv, svx, e0, e1, t0, t1,
                acc_sh, deg_sh):
    c = lax.axis_index("c")
    s = lax.axis_index("s")
    g = (g0, g1, g2, g3)
    gsem = (gs0, gs1, gs2, gs3)
    ssem = (ss0, ss1, ss2, ss3)
    ebufs = (e0, e1)
    tbufs = (t0, t1)
    n_it = _TCH // _NBUF

    pltpu.sync_copy(rows_hbm.at[s], rows_v)
    pltpu.sync_copy(cols_hbm.at[s], cols_v)

    def _fill_ones(i, _):
        ones_v[pl.ds(i * 16, 16)] = jnp.ones((16,), jnp.float32)
        return 0

    lax.fori_loop(0, _CH // 16, _fill_ones, 0)

    def _fill_zero1(i, _):
        sv[pl.ds(i * 16, 16)] = jnp.zeros((16,), jnp.float32)
        return 0

    lax.fori_loop(0, _STR // 16, _fill_zero1, 0)

    def _fill_zero(i, _):
        for l in range(_H // 16):
            g0[i, pl.ds(l * 16, 16)] = jnp.zeros((16,), jnp.float32)
        return 0

    lax.fori_loop(0, _CH, _fill_zero, 0)

    pltpu.sync_copy(sv, deg_sh.at[pl.ds(s * _STR, _STR)])
    plsc.subcore_barrier()

    def _deg_fire(j, _):
        pltpu.async_copy(ones_v, deg_sh.at[rows_v.at[j]], ss0, add=True)
        return 0

    lax.fori_loop(0, _TCH, _deg_fire, 0)

    def _deg_drain(j, _):
        pltpu.make_async_copy(ones_v, deg_sh.at[rows_v.at[j]], ss0).wait()
        return 0

    lax.fori_loop(0, _TCH, _deg_drain, 0)
    plsc.subcore_barrier()

    pltpu.sync_copy(deg_sh.at[pl.ds(s * _STR, _STR)], sv)

    def _rsqrt_stripe(i, _):
        d = sv[pl.ds(i * 16, 16)]
        y = _newton_rsqrt(d)
        sv[pl.ds(i * 16, 16)] = y
        return 0

    lax.fori_loop(0, _STR // 16, _rsqrt_stripe, 0)

    ones16 = jnp.ones((16,), jnp.float32)

    def _splat(i, _):
        s16 = sv[pl.ds(i * 16, 16)]
        for r in range(16):
            svx[i * 16 + r, pl.ds(0, 16)] = s16[r] * ones16
        return 0

    lax.fori_loop(0, _STR // 16, _splat, 0)

    u0_segs = []
    for p in range(_NPASS):
        qbase = (c * _NPASS + p) * _NP + s * _STR
        for qs in range(_NSEG):
            u0_segs.append((qbase + qs * _ESEG, qs * _ESEG))

    def _u0_read(idx):
        i = idx % 2
        hb, _sg = u0_segs[idx]
        return pltpu.async_copy(
            x_hbm.at[pl.ds(hb, _ESEG)], ebufs[i], gsem[i])

    rd = {0: _u0_read(0)}
    wr = {}
    for idx in range(len(u0_segs)):
        i = idx % 2
        hb, sg = u0_segs[idx]
        rd.pop(idx).wait()
        if idx + 1 < len(u0_segs):
            if idx >= 1:
                wr.pop(idx - 1).wait()
            rd[idx + 1] = _u0_read(idx + 1)

        def _scale_u0(b, _, _sg=sg, _i=i):
            f = svx[_sg + b, pl.ds(0, 16)]
            for l in range(_H // 16):
                sl = pl.ds(l * 16, 16)
                ebufs[_i][b, sl] = f * ebufs[_i][b, sl]
            return 0

        lax.fori_loop(0, _ESEG, _scale_u0, 0)
        wr[idx] = pltpu.async_copy(
            ebufs[i], u_hbm.at[pl.ds(hb, _ESEG)], ssem[i])
    wr.pop(len(u0_segs) - 2).wait()
    wr.pop(len(u0_segs) - 1).wait()

    plsc.subcore_barrier()

    for k in range(_K):
        for p in range(_NPASS):
            base = (c * _NPASS + p) * _NP
            if k == 0 and p == 0:
                delta = c * (_NPASS * _NP)
            elif p == 0:
                delta = -_NP
            else:
                delta = _NP

            def _offset(j, _):
                for l in range(_CH // 16):
                    sl = pl.ds(l * 16, 16)
                    cols_v[j, sl] = cols_v[j, sl] + delta
                return 0

            lax.fori_loop(0, _TCH, _offset, 0)
            for z in range(_STR // _CH):
                pltpu.sync_copy(
                    g0, acc_sh.at[pl.ds(s * _STR + z * _CH, _CH)])
            plsc.subcore_barrier()

            for b in range(_NBUF):
                pltpu.async_copy(u_hbm.at[cols_v.at[b]], g[b], gsem[b])

            def _edge_chunks(j, _):
                for b in range(_NBUF):
                    pltpu.make_async_copy(
                        u_hbm.at[cols_v.at[_NBUF * j + b]], g[b], gsem[b]
                    ).wait()
                    pltpu.async_copy(
                        g[b], acc_sh.at[rows_v.at[_NBUF * j + b]], ssem[b],
                        add=True)

                @pl.when(j < n_it - 1)
                def _prefetch():
                    for b in range(_NBUF):
                        pltpu.make_async_copy(
                            g[b], acc_sh.at[rows_v.at[_NBUF * j + b]],
                            ssem[b]).wait()
                        pltpu.async_copy(
                            u_hbm.at[cols_v.at[_NBUF * (j + 1) + b]], g[b],
                            gsem[b])

                return 0

            lax.fori_loop(0, n_it, _edge_chunks, 0)
            for b in range(_NBUF):
                pltpu.make_async_copy(
                    g[b], acc_sh.at[rows_v.at[_TCH - _NBUF + b]], ssem[b]
                ).wait()
            plsc.subcore_barrier()

            hbase = base + s * _STR
            last = k + 1 == _K

            def _epi_read(qs):
                i = qs % 2
                hb = hbase + qs * _ESEG
                sb = s * _STR + qs * _ESEG
                dt = pltpu.async_copy(
                    acc_sh.at[pl.ds(sb, _ESEG)], tbufs[i], gsem[i])
                if k == 0:
                    de = pltpu.async_copy(
                        x_hbm.at[pl.ds(hb, _ESEG)], ebufs[i], gsem[2 + i])
                else:
                    de = pltpu.async_copy(
                        out_hbm.at[pl.ds(hb, _ESEG)], ebufs[i], gsem[2 + i])
                return (dt, de)

            erd = {0: _epi_read(0)}
            ewr = {}
            for qs in range(_NSEG):
                i = qs % 2
                hb = hbase + qs * _ESEG
                sg = qs * _ESEG
                dt, de = erd.pop(qs)
                dt.wait()
                de.wait()
                if qs + 1 < _NSEG:
                    if qs >= 1:
                        for d in ewr.pop(qs - 1):
                            d.wait()
                    erd[qs + 1] = _epi_read(qs + 1)

                if not last:
                    def _scale_mid(b, _, _sg=sg, _i=i):
                        f = svx[_sg + b, pl.ds(0, 16)]
                        eb = ebufs[_i]
                        tb = tbufs[_i]
                        for l in range(_H // 16):
                            sl = pl.ds(l * 16, 16)
                            st = f * tb[b, sl]
                            eb[b, sl] = eb[b, sl] + st
                            tb[b, sl] = f * st
                        return 0

                    lax.fori_loop(0, _ESEG, _scale_mid, 0)
                    ewr[qs] = (
                        pltpu.async_copy(
                            ebufs[i], out_hbm.at[pl.ds(hb, _ESEG)],
                            ssem[i]),
                        pltpu.async_copy(
                            tbufs[i], u_hbm.at[pl.ds(hb, _ESEG)],
                            ssem[2 + i]),
                    )
                else:
                    def _scale_last(b, _, _sg=sg, _i=i):
                        f = svx[_sg + b, pl.ds(0, 16)]
                        eb = ebufs[_i]
                        tb = tbufs[_i]
                        for l in range(_H // 16):
                            sl = pl.ds(l * 16, 16)
                            eb[b, sl] = 0.25 * (
                                eb[b, sl] + f * tb[b, sl])
                        return 0

                    lax.fori_loop(0, _ESEG, _scale_last, 0)
                    ewr[qs] = (
                        pltpu.async_copy(
                            ebufs[i], out_hbm.at[pl.ds(hb, _ESEG)],
                            ssem[i]),
                    )
            for qs in (_NSEG - 2, _NSEG - 1):
                for d in ewr.pop(qs):
                    d.wait()

            if k + 1 < _K or p + 1 < _NPASS:
                lax.fori_loop(0, _CH, _fill_zero, 0)
                plsc.subcore_barrier()


def _fused_call(x_flat, rows_m, cols_m):
    return pl.kernel(
        _fused_body,
        out_type=[
            jax.ShapeDtypeStruct((_ROWS, _H), jnp.float32),
            jax.ShapeDtypeStruct((_ROWS, _H), jnp.float32),
        ],
        mesh=_mesh,
        compiler_params=_sc_params,
        scratch_types=[
            pltpu.VMEM((_TCH, _CH), jnp.int32),
            pltpu.VMEM((_TCH, _CH), jnp.int32),
        ] + [pltpu.VMEM((_CH, _H), jnp.float32)] * _NBUF
        + [pltpu.SemaphoreType.DMA] * (2 * _NBUF)
        + [
            pltpu.VMEM((_CH,), jnp.float32),
            pltpu.VMEM((_STR,), jnp.float32),
            pltpu.VMEM((_STR, 16), jnp.float32),
            pltpu.VMEM((_ESEG, _H), jnp.float32),
            pltpu.VMEM((_ESEG, _H), jnp.float32),
            pltpu.VMEM((_ESEG, _H), jnp.float32),
            pltpu.VMEM((_ESEG, _H), jnp.float32),
            pltpu.VMEM_SHARED((_NP, _H), jnp.float32),
            pltpu.VMEM_SHARED((_NP,), jnp.float32),
        ],
    )(x_flat, rows_m, cols_m)


def kernel(x, edge_index):
    rows = edge_index[0]
    cols = edge_index[1]
    pad = _N + (jnp.arange(_EP - _E, dtype=jnp.int32) % (_NP - _N))
    rows_p = jnp.concatenate([rows, pad])
    cols_p = jnp.concatenate([cols, pad])
    rows_m = rows_p.reshape(_NS, _TCH, _CH)
    cols_m = cols_p.reshape(_NS, _TCH, _CH)
    x_flat = (
        jnp.pad(x, ((0, _NP - _N), (0, 0)))
        .reshape(_NP, _Q, _H)
        .transpose(1, 0, 2)
        .reshape(_ROWS, _H)
    )
    out_flat, _ = _fused_call(x_flat, rows_m, cols_m)
    return (
        out_flat.reshape(_Q, _NP, _H)
        .transpose(1, 0, 2)
        .reshape(_NP, _D)[:_N]
    )


# speedup vs baseline: 12.2378x; 1.1201x over previous
import jax
import jax.numpy as jnp
from jax import lax
from jax.experimental import pallas as pl
from jax.experimental.pallas import tpu as pltpu
from jax.experimental.pallas import tpu_sc as plsc

_N = 10000
_NP = 10240
_E = 160000
_EP = 163840
_D = 256
_H = 64
_Q = 4
_NC = 2
_NS = 16
_NPASS = _Q // _NC
_K = 3
_CH = 128
_TCH = _EP // _NS // _CH
_STR = _NP // _NS
_ESEG = 80
_NSEG = _STR // _ESEG
_ROWS = _Q * _NP
_NBUF = 4

_mesh = plsc.VectorSubcoreMesh(core_axis_name="c", subcore_axis_name="s")
_sc_params = pltpu.CompilerParams(use_tc_tiling_on_sc=False)


def _newton_rsqrt(d):
    y = lax.bitcast_convert_type(
        jnp.int32(0x5F3759DF) - lax.shift_right_logical(
            lax.bitcast_convert_type(d, jnp.int32), jnp.int32(1)),
        jnp.float32)
    for _ in range(3):
        y = y * (1.5 - 0.5 * d * y * y)
    return jnp.where(d > 0.0, y, jnp.zeros_like(y))


def _fused_body(x_hbm, rows_hbm, cols_hbm,
                out_hbm, u_hbm,
                rows_v, cols_v,
                g0, g1, g2, g3,
                gs0, gs1, gs2, gs3,
                ss0, ss1, ss2, ss3,
                ones_v, sv, svx, e0, e1, t0, t1,
                acc_sh, deg_sh):
    c = lax.axis_index("c")
    s = lax.axis_index("s")
    g = (g0, g1, g2, g3)
    gsem = (gs0, gs1, gs2, gs3)
    ssem = (ss0, ss1, ss2, ss3)
    ebufs = (e0, e1)
    tbufs = (t0, t1)
    n_it = _TCH // _NBUF

    pltpu.sync_copy(rows_hbm.at[s], rows_v)
    pltpu.sync_copy(cols_hbm.at[s], cols_v)

    def _fill_ones(i, _):
        ones_v[pl.ds(i * 16, 16)] = jnp.ones((16,), jnp.float32)
        return 0

    lax.fori_loop(0, _CH // 16, _fill_ones, 0)

    def _fill_zero1(i, _):
        sv[pl.ds(i * 16, 16)] = jnp.zeros((16,), jnp.float32)
        return 0

    lax.fori_loop(0, _STR // 16, _fill_zero1, 0)

    def _fill_zero(i, _):
        for l in range(_H // 16):
            g0[i, pl.ds(l * 16, 16)] = jnp.zeros((16,), jnp.float32)
        return 0

    lax.fori_loop(0, _CH, _fill_zero, 0)

    pltpu.sync_copy(sv, deg_sh.at[pl.ds(s * _STR, _STR)])
    plsc.subcore_barrier()

    def _deg_fire(j, _):
        pltpu.async_copy(ones_v, deg_sh.at[rows_v.at[j]], ss0, add=True)
        return 0

    lax.fori_loop(0, _TCH, _deg_fire, 0)

    def _deg_drain(j, _):
        pltpu.make_async_copy(ones_v, deg_sh.at[rows_v.at[j]], ss0).wait()
        return 0

    lax.fori_loop(0, _TCH, _deg_drain, 0)
    plsc.subcore_barrier()

    pltpu.sync_copy(deg_sh.at[pl.ds(s * _STR, _STR)], sv)

    def _rsqrt_stripe(i, _):
        d = sv[pl.ds(i * 16, 16)]
        y = _newton_rsqrt(d)
        sv[pl.ds(i * 16, 16)] = y
        return 0

    lax.fori_loop(0, _STR // 16, _rsqrt_stripe, 0)

    ones16 = jnp.ones((16,), jnp.float32)

    def _splat(i, _):
        s16 = sv[pl.ds(i * 16, 16)]
        for r in range(16):
            svx[i * 16 + r, pl.ds(0, 16)] = s16[r] * ones16
        return 0

    lax.fori_loop(0, _STR // 16, _splat, 0)

    u0_segs = []
    for p in range(_NPASS):
        qq = c * _NPASS + p
        qbase = qq * _NP + s * _STR
        for qs in range(_NSEG):
            u0_segs.append((qbase + qs * _ESEG, qs * _ESEG, qq))

    def _u0_read(idx):
        i = idx % 2
        _hb, _sg, _qq = u0_segs[idx]
        nb = s * _STR + _sg
        return pltpu.async_copy(
            x_hbm.at[pl.ds(nb, _ESEG), pl.ds(_qq * _H, _H)],
            ebufs[i], gsem[i])

    rd = {0: _u0_read(0)}
    wr = {}
    for idx in range(len(u0_segs)):
        i = idx % 2
        hb, sg, qq = u0_segs[idx]
        rd.pop(idx).wait()
        if idx + 1 < len(u0_segs):
            if idx >= 1:
                wr.pop(idx - 1).wait()
            rd[idx + 1] = _u0_read(idx + 1)

        def _scale_u0(b, _, _sg=sg, _i=i):
            f = svx[_sg + b, pl.ds(0, 16)]
            for l in range(_H // 16):
                sl = pl.ds(l * 16, 16)
                ebufs[_i][b, sl] = f * ebufs[_i][b, sl]
            return 0

        lax.fori_loop(0, _ESEG, _scale_u0, 0)
        wr[idx] = pltpu.async_copy(
            ebufs[i], u_hbm.at[pl.ds(hb, _ESEG)], ssem[i])
    wr.pop(len(u0_segs) - 2).wait()
    wr.pop(len(u0_segs) - 1).wait()

    plsc.subcore_barrier()

    for k in range(_K):
        for p in range(_NPASS):
            base = (c * _NPASS + p) * _NP
            if k == 0 and p == 0:
                delta = c * (_NPASS * _NP)
            elif p == 0:
                delta = -_NP
            else:
                delta = _NP

            def _offset(j, _):
                for l in range(_CH // 16):
                    sl = pl.ds(l * 16, 16)
                    cols_v[j, sl] = cols_v[j, sl] + delta
                return 0

            lax.fori_loop(0, _TCH, _offset, 0)
            for z in range(_STR // _CH):
                pltpu.sync_copy(
                    g0, acc_sh.at[pl.ds(s * _STR + z * _CH, _CH)])
            plsc.subcore_barrier()

            for b in range(_NBUF):
                pltpu.async_copy(u_hbm.at[cols_v.at[b]], g[b], gsem[b])

            def _edge_chunks(j, _):
                for b in range(_NBUF):
                    pltpu.make_async_copy(
                        u_hbm.at[cols_v.at[_NBUF * j + b]], g[b], gsem[b]
                    ).wait()
                    pltpu.async_copy(
                        g[b], acc_sh.at[rows_v.at[_NBUF * j + b]], ssem[b],
                        add=True)

                @pl.when(j < n_it - 1)
                def _prefetch():
                    for b in range(_NBUF):
                        pltpu.make_async_copy(
                            g[b], acc_sh.at[rows_v.at[_NBUF * j + b]],
                            ssem[b]).wait()
                        pltpu.async_copy(
                            u_hbm.at[cols_v.at[_NBUF * (j + 1) + b]], g[b],
                            gsem[b])

                return 0

            lax.fori_loop(0, n_it, _edge_chunks, 0)
            for b in range(_NBUF):
                pltpu.make_async_copy(
                    g[b], acc_sh.at[rows_v.at[_TCH - _NBUF + b]], ssem[b]
                ).wait()
            plsc.subcore_barrier()

            hbase = base + s * _STR
            qq = c * _NPASS + p
            last = k + 1 == _K

            def _epi_read(qs):
                i = qs % 2
                sb = s * _STR + qs * _ESEG
                dt = pltpu.async_copy(
                    acc_sh.at[pl.ds(sb, _ESEG)], tbufs[i], gsem[i])
                if k == 0:
                    de = pltpu.async_copy(
                        x_hbm.at[pl.ds(sb, _ESEG), pl.ds(qq * _H, _H)],
                        ebufs[i], gsem[2 + i])
                else:
                    de = pltpu.async_copy(
                        out_hbm.at[pl.ds(sb, _ESEG), pl.ds(qq * _H, _H)],
                        ebufs[i], gsem[2 + i])
                return (dt, de)

            erd = {0: _epi_read(0)}
            ewr = {}
            for qs in range(_NSEG):
                i = qs % 2
                sb = s * _STR + qs * _ESEG
                sg = qs * _ESEG
                dt, de = erd.pop(qs)
                dt.wait()
                de.wait()
                if qs + 1 < _NSEG:
                    if qs >= 1:
                        for d in ewr.pop(qs - 1):
                            d.wait()
                    erd[qs + 1] = _epi_read(qs + 1)

                if not last:
                    def _scale_mid(b, _, _sg=sg, _i=i):
                        f = svx[_sg + b, pl.ds(0, 16)]
                        eb = ebufs[_i]
                        tb = tbufs[_i]
                        for l in range(_H // 16):
                            sl = pl.ds(l * 16, 16)
                            st = f * tb[b, sl]
                            eb[b, sl] = eb[b, sl] + st
                            tb[b, sl] = f * st
                        return 0

                    lax.fori_loop(0, _ESEG, _scale_mid, 0)
                    ewr[qs] = (
                        pltpu.async_copy(
                            ebufs[i],
                            out_hbm.at[pl.ds(sb, _ESEG),
                                       pl.ds(qq * _H, _H)],
                            ssem[i]),
                        pltpu.async_copy(
                            tbufs[i], u_hbm.at[pl.ds(base + sb, _ESEG)],
                            ssem[2 + i]),
                    )
                else:
                    def _scale_last(b, _, _sg=sg, _i=i):
                        f = svx[_sg + b, pl.ds(0, 16)]
                        eb = ebufs[_i]
                        tb = tbufs[_i]
                        for l in range(_H // 16):
                            sl = pl.ds(l * 16, 16)
                            eb[b, sl] = 0.25 * (
                                eb[b, sl] + f * tb[b, sl])
                        return 0

                    lax.fori_loop(0, _ESEG, _scale_last, 0)
                    ewr[qs] = (
                        pltpu.async_copy(
                            ebufs[i],
                            out_hbm.at[pl.ds(sb, _ESEG),
                                       pl.ds(qq * _H, _H)],
                            ssem[i]),
                    )
            for qs in (_NSEG - 2, _NSEG - 1):
                for d in ewr.pop(qs):
                    d.wait()

            if k + 1 < _K or p + 1 < _NPASS:
                lax.fori_loop(0, _CH, _fill_zero, 0)
                plsc.subcore_barrier()


def _fused_call(x_flat, rows_m, cols_m):
    return pl.kernel(
        _fused_body,
        out_type=[
            jax.ShapeDtypeStruct((_NP, _D), jnp.float32),
            jax.ShapeDtypeStruct((_ROWS, _H), jnp.float32),
        ],
        mesh=_mesh,
        compiler_params=_sc_params,
        scratch_types=[
            pltpu.VMEM((_TCH, _CH), jnp.int32),
            pltpu.VMEM((_TCH, _CH), jnp.int32),
        ] + [pltpu.VMEM((_CH, _H), jnp.float32)] * _NBUF
        + [pltpu.SemaphoreType.DMA] * (2 * _NBUF)
        + [
            pltpu.VMEM((_CH,), jnp.float32),
            pltpu.VMEM((_STR,), jnp.float32),
            pltpu.VMEM((_STR, 16), jnp.float32),
            pltpu.VMEM((_ESEG, _H), jnp.float32),
            pltpu.VMEM((_ESEG, _H), jnp.float32),
            pltpu.VMEM((_ESEG, _H), jnp.float32),
            pltpu.VMEM((_ESEG, _H), jnp.float32),
            pltpu.VMEM_SHARED((_NP, _H), jnp.float32),
            pltpu.VMEM_SHARED((_NP,), jnp.float32),
        ],
    )(x_flat, rows_m, cols_m)


def kernel(x, edge_index):
    rows = edge_index[0]
    cols = edge_index[1]
    pad = _N + (jnp.arange(_EP - _E, dtype=jnp.int32) % (_NP - _N))
    rows_p = jnp.concatenate([rows, pad])
    cols_p = jnp.concatenate([cols, pad])
    rows_m = rows_p.reshape(_NS, _TCH, _CH)
    cols_m = cols_p.reshape(_NS, _TCH, _CH)
    x_pad = jnp.pad(x, ((0, _NP - _N), (0, 0)))
    out2d, _ = _fused_call(x_pad, rows_m, cols_m)
    return out2d[:_N]
